# baseline (speedup 1.0000x reference)
import functools

import jax
import jax.numpy as jnp
from jax import lax
from jax.experimental import pallas as pl
from jax.experimental.pallas import tpu as pltpu
from jax.experimental.pallas import tpu_sc as plsc


def _sc_gather(table, idx):
    NQ, D = table.shape
    B = idx.shape[0]
    table2 = table.reshape(NQ // 2, 2 * D)
    info = plsc.get_sparse_core_info()
    nc, ns, L = info.num_cores, info.num_subcores, info.num_lanes
    nw = nc * ns
    n = B // nw
    mesh = plsc.VectorSubcoreMesh(core_axis_name="c", subcore_axis_name="s")

    @functools.partial(
        pl.kernel,
        mesh=mesh,
        out_type=jax.ShapeDtypeStruct((B, 2 * D), jnp.float32),
        scratch_types=[
            pltpu.VMEM((n,), jnp.int32),
            pltpu.VMEM((n,), jnp.int32),
            pltpu.VMEM((n, 2 * D), jnp.float32),
            pltpu.SemaphoreType.DMA,
        ],
    )
    def k(table_hbm, idx_hbm, out_hbm, idx_v, idx2_v, rows, sem):
        wid = lax.axis_index("s") * nc + lax.axis_index("c")
        base = wid * n
        pltpu.sync_copy(idx_hbm.at[pl.ds(base, n)], idx_v)
        for j in range(n // L):
            idx2_v[pl.ds(j * L, L)] = jnp.right_shift(idx_v[pl.ds(j * L, L)],
                                                      1)
        pltpu.async_copy(table_hbm.at[idx2_v], rows, sem).wait()
        pltpu.sync_copy(rows, out_hbm.at[pl.ds(base, n)])

    return k(table2, idx)


def _tc_body(t_ref, q2_ref, par_ref, df_ref, S_ref, Wd_ref, bd_ref, W1q_ref,
             W1m_ref, W1a_ref, w1p_ref, b1_ref, W2_ref, b2_ref, e_ref, p_ref):
    mask = (t_ref[...] != 0).astype(jnp.float32)
    cnt = jnp.maximum(jnp.sum(mask, axis=1, keepdims=True), 1.0)
    mu = lax.dot_general(mask, S_ref[...], (((1,), (0,)), ((), ())),
                         preferred_element_type=jnp.float32) / cnt
    q2 = q2_ref[...]
    D = q2.shape[1] // 2
    parT = jnp.transpose(par_ref[...])
    q = jnp.where(parT > 0.5, q2[:, D:], q2[:, :D])
    a = jnp.dot(df_ref[...], Wd_ref[...],
                preferred_element_type=jnp.float32) + bd_ref[...]
    p12 = jnp.sum(q * mu, axis=-1, keepdims=True)
    p13 = jnp.sum(q * a, axis=-1, keepdims=True)
    p23 = jnp.sum(mu * a, axis=-1, keepdims=True)
    z = (jnp.dot(q, W1q_ref[...], preferred_element_type=jnp.float32)
         + jnp.dot(mu, W1m_ref[...], preferred_element_type=jnp.float32)
         + jnp.dot(a, W1a_ref[...], preferred_element_type=jnp.float32)
         + p12 * w1p_ref[0:1, :] + p13 * w1p_ref[1:2, :] + p23 * w1p_ref[2:3, :]
         + b1_ref[...])
    e = jnp.maximum(z, 0.0)
    e_ref[...] = e
    p_ref[...] = jnp.dot(e, W2_ref[...],
                         preferred_element_type=jnp.float32) + b2_ref[...]


def kernel(questions, question_skill_targets, difficulty_feats, Q_table,
           S_table, W_diff, b_diff, W1, b1, W2, b2):
    B, NT = question_skill_targets.shape
    D = Q_table.shape[1]
    H = W1.shape[1]
    qi = questions.astype(jnp.int32)
    q2 = _sc_gather(Q_table, qi)
    par = jnp.bitwise_and(qi, 1).astype(jnp.float32).reshape(1, B)

    bB = 512
    grid = (B // bB,)
    W1q, W1m, W1a, w1p = W1[0:D], W1[D:2 * D], W1[2 * D:3 * D], W1[3 * D:]
    bd2 = b_diff.reshape(1, D)
    b12 = b1.reshape(1, H)
    b22 = b2.reshape(1, 1)

    e, p = pl.pallas_call(
        _tc_body,
        grid=grid,
        in_specs=[
            pl.BlockSpec((bB, NT), lambda i: (i, 0)),
            pl.BlockSpec((bB, 2 * D), lambda i: (i, 0)),
            pl.BlockSpec((1, bB), lambda i: (0, i)),
            pl.BlockSpec((bB, difficulty_feats.shape[1]), lambda i: (i, 0)),
            pl.BlockSpec((NT, D), lambda i: (0, 0)),
            pl.BlockSpec(W_diff.shape, lambda i: (0, 0)),
            pl.BlockSpec((1, D), lambda i: (0, 0)),
            pl.BlockSpec((D, H), lambda i: (0, 0)),
            pl.BlockSpec((D, H), lambda i: (0, 0)),
            pl.BlockSpec((D, H), lambda i: (0, 0)),
            pl.BlockSpec((3, H), lambda i: (0, 0)),
            pl.BlockSpec((1, H), lambda i: (0, 0)),
            pl.BlockSpec((H, 1), lambda i: (0, 0)),
            pl.BlockSpec((1, 1), lambda i: (0, 0)),
        ],
        out_specs=[
            pl.BlockSpec((bB, H), lambda i: (i, 0)),
            pl.BlockSpec((bB, 1), lambda i: (i, 0)),
        ],
        out_shape=[
            jax.ShapeDtypeStruct((B, H), jnp.float32),
            jax.ShapeDtypeStruct((B, 1), jnp.float32),
        ],
        compiler_params=pltpu.CompilerParams(
            dimension_semantics=("arbitrary",),
        ),
    )(question_skill_targets, q2, par, difficulty_feats, S_table, W_diff, bd2,
      W1q, W1m, W1a, w1p, b12, W2, b22)
    return (e, p)


# speedup vs baseline: 1.9303x; 1.9303x over previous
import functools

import jax
import jax.numpy as jnp
from jax import lax
from jax.experimental import pallas as pl
from jax.experimental.pallas import tpu as pltpu
from jax.experimental.pallas import tpu_sc as plsc


def _sc_gather(table, idx):
    NQ, D = table.shape
    B = idx.shape[0]
    table3 = table.reshape(NQ // 8, 8, D)
    info = plsc.get_sparse_core_info()
    nc, ns, L = info.num_cores, info.num_subcores, info.num_lanes
    nw = nc * ns
    n = B // nw
    K = 16
    nch = n // K
    mesh = plsc.VectorSubcoreMesh(core_axis_name="c", subcore_axis_name="s")

    @functools.partial(
        pl.kernel,
        mesh=mesh,
        out_type=jax.ShapeDtypeStruct((B, D), jnp.float32),
        scratch_types=[
            pltpu.VMEM((n,), jnp.int32),
            pltpu.VMEM((2, K, 8, D), jnp.float32),
            pltpu.VMEM((n, D), jnp.float32),
            pltpu.SemaphoreType.DMA,
            pltpu.SemaphoreType.DMA,
        ],
        compiler_params=pltpu.CompilerParams(needs_layout_passes=False),
    )
    def k(table_hbm, idx_hbm, out_hbm, idx_v, slabs, rows, sem0, sem1):
        wid = lax.axis_index("s") * nc + lax.axis_index("c")
        base = wid * n
        pltpu.sync_copy(idx_hbm.at[pl.ds(base, n)], idx_v)
        sems = (sem0, sem1)

        def fire(c, bank, sem):
            qv = idx_v[pl.ds(c * K, L)]
            slabv = jnp.right_shift(qv, 3)
            for j in range(K):
                pltpu.async_copy(table_hbm.at[slabv[j]],
                                 slabs.at[bank, j], sem)

        def drain_extract(c, bank, sem):
            for j in range(K):
                pltpu.make_async_copy(table_hbm.at[0], slabs.at[bank, j],
                                      sem).wait()
            qv = idx_v[pl.ds(c * K, L)]
            subv = jnp.bitwise_and(qv, 7)
            r0 = c * K
            for j in range(K):
                sub = subv[j]
                for cc in range(D // L):
                    rows[r0 + j, pl.ds(cc * L, L)] = slabs[bank, j, sub,
                                                           pl.ds(cc * L, L)]

        fire(0, 0, sem0)

        def body(h, _):
            c0 = 2 * h
            fire(c0 + 1, 1, sem1)
            drain_extract(c0, 0, sem0)
            nxt = lax.rem(c0 + 2, nch)
            fire(nxt, 0, sem0)
            drain_extract(c0 + 1, 1, sem1)
            return 0

        lax.fori_loop(0, nch // 2, body, 0)
        for j in range(K):
            pltpu.make_async_copy(table_hbm.at[0], slabs.at[0, j],
                                  sem0).wait()
        pltpu.sync_copy(rows, out_hbm.at[pl.ds(base, n)])

    return k(table3, idx)


def _tc_a(t_ref, df_ref, S_ref, Wd_ref, bd_ref, W1m_ref, W1a_ref, w1p_ref,
          b1_ref, E1_ref, mu_ref, a_ref):
    mask = (t_ref[...] != 0).astype(jnp.float32)
    cnt = jnp.maximum(jnp.sum(mask, axis=1, keepdims=True), 1.0)
    mu = lax.dot_general(mask, S_ref[...], (((1,), (0,)), ((), ())),
                         preferred_element_type=jnp.float32) / cnt
    a = jnp.dot(df_ref[...], Wd_ref[...],
                preferred_element_type=jnp.float32) + bd_ref[...]
    p23 = jnp.sum(mu * a, axis=-1, keepdims=True)
    E1 = (jnp.dot(mu, W1m_ref[...], preferred_element_type=jnp.float32)
          + jnp.dot(a, W1a_ref[...], preferred_element_type=jnp.float32)
          + p23 * w1p_ref[2:3, :] + b1_ref[...])
    E1_ref[...] = E1
    mu_ref[...] = mu
    a_ref[...] = a


def _tc_b(q_ref, E1_ref, mu_ref, a_ref, W1q_ref, w1p_ref, W2_ref, b2_ref,
          e_ref, p_ref):
    q = q_ref[...]
    mu = mu_ref[...]
    a = a_ref[...]
    p12 = jnp.sum(q * mu, axis=-1, keepdims=True)
    p13 = jnp.sum(q * a, axis=-1, keepdims=True)
    z = (E1_ref[...] + jnp.dot(q, W1q_ref[...],
                               preferred_element_type=jnp.float32)
         + p12 * w1p_ref[0:1, :] + p13 * w1p_ref[1:2, :])
    e = jnp.maximum(z, 0.0)
    e_ref[...] = e
    p_ref[...] = jnp.dot(e, W2_ref[...],
                         preferred_element_type=jnp.float32) + b2_ref[...]


def kernel(questions, question_skill_targets, difficulty_feats, Q_table,
           S_table, W_diff, b_diff, W1, b1, W2, b2):
    B, NT = question_skill_targets.shape
    DF = difficulty_feats.shape[1]
    D = Q_table.shape[1]
    H = W1.shape[1]
    qi = questions.astype(jnp.int32)
    q = _sc_gather(Q_table, qi)

    bB = 512
    grid = (B // bB,)
    W1q, W1m, W1a, w1p = W1[0:D], W1[D:2 * D], W1[2 * D:3 * D], W1[3 * D:]
    bd2 = b_diff.reshape(1, D)
    b12 = b1.reshape(1, H)
    b22 = b2.reshape(1, 1)

    full = lambda i: (0, 0)
    rows = lambda i: (i, 0)

    E1, mu, a = pl.pallas_call(
        _tc_a,
        grid=grid,
        in_specs=[
            pl.BlockSpec((bB, NT), rows),
            pl.BlockSpec((bB, DF), rows),
            pl.BlockSpec((NT, D), full),
            pl.BlockSpec((DF, D), full),
            pl.BlockSpec((1, D), full),
            pl.BlockSpec((D, H), full),
            pl.BlockSpec((D, H), full),
            pl.BlockSpec((3, H), full),
            pl.BlockSpec((1, H), full),
        ],
        out_specs=[
            pl.BlockSpec((bB, H), rows),
            pl.BlockSpec((bB, D), rows),
            pl.BlockSpec((bB, D), rows),
        ],
        out_shape=[
            jax.ShapeDtypeStruct((B, H), jnp.float32),
            jax.ShapeDtypeStruct((B, D), jnp.float32),
            jax.ShapeDtypeStruct((B, D), jnp.float32),
        ],
        compiler_params=pltpu.CompilerParams(
            dimension_semantics=("arbitrary",),
        ),
    )(question_skill_targets, difficulty_feats, S_table, W_diff, bd2,
      W1m, W1a, w1p, b12)

    e, p = pl.pallas_call(
        _tc_b,
        grid=grid,
        in_specs=[
            pl.BlockSpec((bB, D), rows),
            pl.BlockSpec((bB, H), rows),
            pl.BlockSpec((bB, D), rows),
            pl.BlockSpec((bB, D), rows),
            pl.BlockSpec((D, H), full),
            pl.BlockSpec((3, H), full),
            pl.BlockSpec((H, 1), full),
            pl.BlockSpec((1, 1), full),
        ],
        out_specs=[
            pl.BlockSpec((bB, H), rows),
            pl.BlockSpec((bB, 1), rows),
        ],
        out_shape=[
            jax.ShapeDtypeStruct((B, H), jnp.float32),
            jax.ShapeDtypeStruct((B, 1), jnp.float32),
        ],
        compiler_params=pltpu.CompilerParams(
            dimension_semantics=("arbitrary",),
        ),
    )(q, E1, mu, a, W1q, w1p, W2, b22)
    return (e, p)
